# baseline (speedup 1.0000x reference)
import functools

import jax
import jax.numpy as jnp
from jax import lax
from jax.experimental import pallas as pl
from jax.experimental.pallas import tpu as pltpu
from jax.experimental.pallas import tpu_sc as plsc

_T = 4096
_D = 4096
_E = 64
_CAP = 128
_TB1 = 512
_TB2 = 256

_NC = 2
_NS = 16
_NW = _NC * _NS
_L = 16
_FLAT = _T * _E * _CAP
_PER_W = _FLAT // _NW
_CHUNK = 32768
_NCH = _PER_W // _CHUNK
_TPW = _T // _NW

_sc_mesh = plsc.VectorSubcoreMesh(core_axis_name="c", subcore_axis_name="s")


def _gate_kernel(x_ref, wg_ref, gum_ref,
                 idx1_ref, idx2_ref, g1_ref, g2_ref, loc1_ref, loc2_ref,
                 gsum_ref, expc_ref, cnt1_ref, cnt2_ref, tri_ref):
  i = pl.program_id(0)
  nb = pl.num_programs(0)

  @pl.when(i == 0)
  def _init():
    cnt1_ref[...] = jnp.zeros((1, _E), jnp.float32)
    cnt2_ref[...] = jnp.zeros((1, _E), jnp.float32)
    gsum_ref[...] = jnp.zeros((1, _E), jnp.float32)
    r = lax.broadcasted_iota(jnp.int32, (_TB1, _TB1), 0)
    c = lax.broadcasted_iota(jnp.int32, (_TB1, _TB1), 1)
    tri_ref[...] = (r >= c).astype(jnp.float32)

  logits = jnp.dot(x_ref[...], wg_ref[...],
                   preferred_element_type=jnp.float32)
  m = jnp.max(logits, axis=1, keepdims=True)
  ex = jnp.exp(logits - m)
  gates = ex / jnp.sum(ex, axis=1, keepdims=True)

  iota_e = lax.broadcasted_iota(jnp.int32, (_TB1, _E), 1)
  gmax = jnp.max(gates, axis=1, keepdims=True)
  idx1 = jnp.min(jnp.where(gates == gmax, iota_e, _E),
                 axis=1, keepdims=True)
  lw = logits + gum_ref[...]
  lw2 = jnp.where(iota_e == idx1, -jnp.inf, lw)
  m2 = jnp.max(lw2, axis=1, keepdims=True)
  idx2 = jnp.min(jnp.where(lw2 == m2, iota_e, _E),
                 axis=1, keepdims=True)
  mask1 = (iota_e == idx1).astype(jnp.float32)
  mask2 = (iota_e == idx2).astype(jnp.float32)
  g2 = jnp.dot(gates * mask2, jnp.ones((_E, 1), jnp.float32),
               preferred_element_type=jnp.float32)

  tri = tri_ref[...]
  csum1 = jnp.dot(tri, mask1, preferred_element_type=jnp.float32)
  csum2 = jnp.dot(tri, mask2, preferred_element_type=jnp.float32)
  loc1 = csum1 - 1.0 + cnt1_ref[...]
  loc2 = csum2 - 1.0 + cnt2_ref[...]
  ones_e = jnp.ones((_E, 1), jnp.float32)
  ones_t = jnp.ones((1, _TB1), jnp.float32)
  loc1_s = jnp.dot(mask1 * loc1, ones_e, preferred_element_type=jnp.float32)
  loc2_s = jnp.dot(mask2 * loc2, ones_e, preferred_element_type=jnp.float32)

  cnt1_ref[...] += jnp.dot(ones_t, mask1, preferred_element_type=jnp.float32)
  cnt2_ref[...] += jnp.dot(ones_t, mask2, preferred_element_type=jnp.float32)
  gsum_ref[...] += jnp.dot(ones_t, gates, preferred_element_type=jnp.float32)

  idx1_ref[...] = idx1.reshape(_TB1)
  idx2_ref[...] = idx2.reshape(_TB1)
  g1_ref[...] = gmax.reshape(_TB1)
  g2_ref[...] = g2.reshape(_TB1)
  loc1_ref[...] = loc1_s.astype(jnp.int32).reshape(_TB1)
  loc2_ref[...] = loc2_s.astype(jnp.int32).reshape(_TB1)

  @pl.when(i == nb - 1)
  def _fin():
    expc_ref[...] = cnt1_ref[...].astype(jnp.int32)


def _disp_kernel(idx1_ref, idx2_ref, loc1_ref, loc2_ref, g1_ref, g2_ref,
                 expc_ref, gsum_ref, disp_ref, laux_ref, loc2f_ref,
                 key_ref):
  i = pl.program_id(0)

  @pl.when(i == 0)
  def _mkkey():
    ke = lax.broadcasted_iota(jnp.int32, (_E, _CAP), 0)
    kc = lax.broadcasted_iota(jnp.int32, (_E, _CAP), 1)
    key_ref[...] = (ke * _CAP + kc).astype(jnp.int16)
  idx1 = idx1_ref[...].reshape(_TB2, 1)
  idx2 = idx2_ref[...].reshape(_TB2, 1)
  loc1 = loc1_ref[...].reshape(_TB2, 1)
  loc2p = loc2_ref[...].reshape(_TB2, 1)
  g1 = g1_ref[...].reshape(_TB2, 1)
  g2 = g2_ref[...].reshape(_TB2, 1)
  expc = expc_ref[...]

  iota_e = lax.broadcasted_iota(jnp.int32, (_TB2, _E), 1)
  totb = jnp.broadcast_to(expc, (_TB2, _E))
  tot2 = jnp.sum(jnp.where(iota_e == idx2, totb, 0),
                 axis=1, keepdims=True)
  loc2 = loc2p + tot2
  loc2f_ref[...] = loc2.reshape(_TB2)

  keep1 = loc1 < _CAP
  keep2 = loc2 < _CAP
  g1k = jnp.where(keep1, g1, 0.0)
  g2k = jnp.where(keep2, g2, 0.0)
  nz1 = g1k != 0.0
  nz2 = g2k != 0.0

  k1 = jnp.where(nz1, idx1 * _CAP + loc1, -1).astype(jnp.int16)
  k2 = jnp.where(nz2, idx2 * _CAP + loc2, -1).astype(jnp.int16)
  key = jnp.broadcast_to(key_ref[...][None], (_TB2, _E, _CAP))
  disp_ref[...] = ((key == k1[:, :, None]) |
                   (key == k2[:, :, None])).astype(jnp.int8)

  @pl.when(i == 0)
  def _laux():
    me = gsum_ref[...] * (1.0 / _T)
    ce = expc.astype(jnp.float32) * (1.0 / _T)
    laux_ref[...] = (jnp.sum(me * ce) * float(_E)).reshape(1, 1)


@functools.partial(
    pl.kernel, mesh=_sc_mesh,
    out_type=jax.ShapeDtypeStruct((_FLAT,), jnp.float32),
    scratch_types=[pltpu.VMEM((_CHUNK,), jnp.float32),
                   pltpu.SemaphoreType.DMA],
)
def _sc_memset(zsrc_hbm, out_hbm, zbuf, sem):
  wid = lax.axis_index("s") * _NC + lax.axis_index("c")
  base = wid * _PER_W
  pltpu.sync_copy(zsrc_hbm, zbuf)

  pltpu.async_copy(zbuf, out_hbm.at[pl.ds(base, _CHUNK)], sem)

  def body(ch, carry):
    @pl.when(ch < _NCH - 1)
    def _start_next():
      pltpu.async_copy(
          zbuf, out_hbm.at[pl.ds(base + (ch + 1) * _CHUNK, _CHUNK)], sem)
    pltpu.make_async_copy(
        zbuf, out_hbm.at[pl.ds(base, _CHUNK)], sem).wait()
    return carry

  lax.fori_loop(0, _NCH, body, 0)


@functools.partial(
    pl.kernel, mesh=_sc_mesh,
    out_type=(),
    scratch_types=[pltpu.VMEM((_TPW,), jnp.int32),
                   pltpu.VMEM((_TPW,), jnp.int32),
                   pltpu.VMEM((_TPW,), jnp.int32),
                   pltpu.VMEM((_TPW,), jnp.int32),
                   pltpu.VMEM((_TPW,), jnp.float32),
                   pltpu.VMEM((_TPW,), jnp.float32),
                   pltpu.VMEM((2, _TPW), jnp.int32),
                   pltpu.VMEM((2, _TPW), jnp.float32),
                   pltpu.SemaphoreType.DMA,
                   pltpu.SemaphoreType.DMA],
)
def _sc_scatter(comb_hbm, idx1_hbm, idx2_hbm, loc1_hbm, loc2_hbm,
                g1_hbm, g2_hbm,
                i1v, i2v, l1v, l2v, g1v, g2v, wv, vv, sem, sem2):
  wid = lax.axis_index("s") * _NC + lax.axis_index("c")
  base = wid * _TPW
  pltpu.sync_copy(idx1_hbm.at[pl.ds(base, _TPW)], i1v)
  pltpu.sync_copy(idx2_hbm.at[pl.ds(base, _TPW)], i2v)
  pltpu.sync_copy(loc1_hbm.at[pl.ds(base, _TPW)], l1v)
  pltpu.sync_copy(loc2_hbm.at[pl.ds(base, _TPW)], l2v)
  pltpu.sync_copy(g1_hbm.at[pl.ds(base, _TPW)], g1v)
  pltpu.sync_copy(g2_hbm.at[pl.ds(base, _TPW)], g2v)

  eps = jnp.float32(jnp.finfo(jnp.float32).eps)
  for j in range(_TPW // _L):
    sl = pl.ds(j * _L, _L)
    i1 = i1v[sl]
    i2 = i2v[sl]
    l1 = l1v[sl]
    l2 = l2v[sl]
    k1 = l1 < _CAP
    k2 = l2 < _CAP
    g1 = jnp.where(k1, g1v[sl], 0.0)
    g2 = jnp.where(k2, g2v[sl], 0.0)
    den = jnp.maximum(g1 + g2, eps)
    tok = base + j * _L + lax.iota(jnp.int32, _L)
    wv[0, sl] = tok * (_E * _CAP) + i1 * _CAP + jnp.where(k1, l1, 0)
    wv[1, sl] = tok * (_E * _CAP) + i2 * _CAP + jnp.where(k2, l2, 0)
    vv[0, sl] = g1 / den
    vv[1, sl] = g2 / den

  c1 = pltpu.async_copy(vv.at[0], comb_hbm.at[wv.at[0]], sem)
  c2 = pltpu.async_copy(vv.at[1], comb_hbm.at[wv.at[1]], sem2)
  c1.wait()
  c2.wait()


def kernel(input, wg):
  x = input.astype(jnp.float32)
  gum = jax.random.gumbel(jax.random.key(42), (_T, _E), jnp.float32)

  zsrc = jnp.zeros((_CHUNK,), jnp.float32)
  comb0 = _sc_memset(zsrc)

  nb1 = _T // _TB1
  vec_i = jax.ShapeDtypeStruct((_T,), jnp.int32)
  vec_f = jax.ShapeDtypeStruct((_T,), jnp.float32)
  row_f = jax.ShapeDtypeStruct((1, _E), jnp.float32)
  row_i = jax.ShapeDtypeStruct((1, _E), jnp.int32)

  vb = pl.BlockSpec((_TB1,), lambda i: (i,))
  rowb = pl.BlockSpec((1, _E), lambda i: (0, 0))

  idx1, idx2, g1, g2, loc1, loc2, gsum, expc = pl.pallas_call(
      _gate_kernel,
      grid=(nb1,),
      in_specs=[
          pl.BlockSpec((_TB1, _D), lambda i: (i, 0)),
          pl.BlockSpec((_D, _E), lambda i: (0, 0)),
          pl.BlockSpec((_TB1, _E), lambda i: (i, 0)),
      ],
      out_specs=[vb, vb, vb, vb, vb, vb, rowb, rowb],
      out_shape=[vec_i, vec_i, vec_f, vec_f, vec_i, vec_i, row_f, row_i],
      scratch_shapes=[pltpu.VMEM((1, _E), jnp.float32),
                      pltpu.VMEM((1, _E), jnp.float32),
                      pltpu.VMEM((_TB1, _TB1), jnp.float32)],
  )(x, wg, gum)

  nb2 = _T // _TB2
  vb2 = pl.BlockSpec((_TB2,), lambda i: (i,))
  rowb2 = pl.BlockSpec((1, _E), lambda i: (0, 0))
  disp, laux, loc2f = pl.pallas_call(
      _disp_kernel,
      grid=(nb2,),
      in_specs=[vb2, vb2, vb2, vb2, vb2, vb2, rowb2, rowb2],
      out_specs=[
          pl.BlockSpec((_TB2, _E, _CAP), lambda i: (i, 0, 0)),
          pl.BlockSpec((1, 1), lambda i: (0, 0)),
          vb2,
      ],
      out_shape=[
          jax.ShapeDtypeStruct((_T, _E, _CAP), jnp.int8),
          jax.ShapeDtypeStruct((1, 1), jnp.float32),
          jax.ShapeDtypeStruct((_T,), jnp.int32),
      ],
      scratch_shapes=[pltpu.VMEM((_E, _CAP), jnp.int16)],
  )(idx1, idx2, loc1, loc2, g1, g2, expc, gsum)

  cref = jax.new_ref(comb0)
  _sc_scatter(cref, idx1, idx2, loc1, loc2f, g1, g2)
  comb = jax.freeze(cref).reshape(_T, _E, _CAP)

  dispb = disp.view(jnp.bool_)
  return (laux.reshape(()), comb, dispb, expc.reshape(_E))


# speedup vs baseline: 1.2026x; 1.2026x over previous
import functools

import jax
import jax.numpy as jnp
from jax import lax
from jax.experimental import pallas as pl
from jax.experimental.pallas import tpu as pltpu
from jax.experimental.pallas import tpu_sc as plsc

_T = 4096
_D = 4096
_E = 64
_CAP = 128
_TB1 = 512
_TB2 = 256

_NC = 2
_NS = 16
_NW = _NC * _NS
_L = 16
_FLAT = _T * _E * _CAP
_PER_W = _FLAT // _NW
_CHUNK = 32768
_NCH = _PER_W // _CHUNK
_TPW = _T // _NW

_sc_mesh = plsc.VectorSubcoreMesh(core_axis_name="c", subcore_axis_name="s")


def _gate_kernel(x_ref, wg_ref, gum_ref,
                 idx1_ref, idx2_ref, g1_ref, g2_ref, loc1_ref, loc2_ref,
                 gsum_ref, expc_ref, cnt1_ref, cnt2_ref, tri_ref):
  i = pl.program_id(0)
  nb = pl.num_programs(0)

  @pl.when(i == 0)
  def _init():
    cnt1_ref[...] = jnp.zeros((1, _E), jnp.float32)
    cnt2_ref[...] = jnp.zeros((1, _E), jnp.float32)
    gsum_ref[...] = jnp.zeros((1, _E), jnp.float32)
    r = lax.broadcasted_iota(jnp.int32, (_TB1, _TB1), 0)
    c = lax.broadcasted_iota(jnp.int32, (_TB1, _TB1), 1)
    tri_ref[...] = (r >= c).astype(jnp.float32)

  logits = jnp.dot(x_ref[...], wg_ref[...],
                   preferred_element_type=jnp.float32)
  m = jnp.max(logits, axis=1, keepdims=True)
  ex = jnp.exp(logits - m)
  gates = ex / jnp.sum(ex, axis=1, keepdims=True)

  iota_e = lax.broadcasted_iota(jnp.int32, (_TB1, _E), 1)
  gmax = jnp.max(gates, axis=1, keepdims=True)
  idx1 = jnp.min(jnp.where(gates == gmax, iota_e, _E),
                 axis=1, keepdims=True)
  lw = logits + gum_ref[...]
  lw2 = jnp.where(iota_e == idx1, -jnp.inf, lw)
  m2 = jnp.max(lw2, axis=1, keepdims=True)
  idx2 = jnp.min(jnp.where(lw2 == m2, iota_e, _E),
                 axis=1, keepdims=True)
  mask1 = (iota_e == idx1).astype(jnp.float32)
  mask2 = (iota_e == idx2).astype(jnp.float32)
  g2 = jnp.dot(gates * mask2, jnp.ones((_E, 1), jnp.float32),
               preferred_element_type=jnp.float32)

  tri = tri_ref[...]
  csum1 = jnp.dot(tri, mask1, preferred_element_type=jnp.float32)
  csum2 = jnp.dot(tri, mask2, preferred_element_type=jnp.float32)
  loc1 = csum1 - 1.0 + cnt1_ref[...]
  loc2 = csum2 - 1.0 + cnt2_ref[...]
  ones_e = jnp.ones((_E, 1), jnp.float32)
  ones_t = jnp.ones((1, _TB1), jnp.float32)
  loc1_s = jnp.dot(mask1 * loc1, ones_e, preferred_element_type=jnp.float32)
  loc2_s = jnp.dot(mask2 * loc2, ones_e, preferred_element_type=jnp.float32)

  cnt1_ref[...] += jnp.dot(ones_t, mask1, preferred_element_type=jnp.float32)
  cnt2_ref[...] += jnp.dot(ones_t, mask2, preferred_element_type=jnp.float32)
  gsum_ref[...] += jnp.dot(ones_t, gates, preferred_element_type=jnp.float32)

  idx1_ref[...] = idx1.reshape(_TB1)
  idx2_ref[...] = idx2.reshape(_TB1)
  g1_ref[...] = gmax.reshape(_TB1)
  g2_ref[...] = g2.reshape(_TB1)
  loc1_ref[...] = loc1_s.astype(jnp.int32).reshape(_TB1)
  loc2_ref[...] = loc2_s.astype(jnp.int32).reshape(_TB1)

  @pl.when(i == nb - 1)
  def _fin():
    expc_ref[...] = cnt1_ref[...].astype(jnp.int32)


def _disp_kernel(idx1_ref, idx2_ref, loc1_ref, loc2_ref, g1_ref, g2_ref,
                 expc_ref, gsum_ref, disp_ref, laux_ref, loc2f_ref):
  i = pl.program_id(0)
  idx1 = idx1_ref[...].reshape(_TB2, 1)
  idx2 = idx2_ref[...].reshape(_TB2, 1)
  loc1 = loc1_ref[...].reshape(_TB2, 1)
  loc2p = loc2_ref[...].reshape(_TB2, 1)
  g1 = g1_ref[...].reshape(_TB2, 1)
  g2 = g2_ref[...].reshape(_TB2, 1)
  expc = expc_ref[...]

  iota_e = lax.broadcasted_iota(jnp.int32, (_TB2, _E), 1)
  totb = jnp.broadcast_to(expc, (_TB2, _E))
  tot2 = jnp.sum(jnp.where(iota_e == idx2, totb, 0),
                 axis=1, keepdims=True)
  loc2 = loc2p + tot2
  loc2f_ref[...] = loc2.reshape(_TB2)

  keep1 = loc1 < _CAP
  keep2 = loc2 < _CAP
  g1k = jnp.where(keep1, g1, 0.0)
  g2k = jnp.where(keep2, g2, 0.0)
  nz1 = g1k != 0.0
  nz2 = g2k != 0.0

  k1 = jnp.where(nz1, idx1 * _CAP + loc1, -1)
  k2 = jnp.where(nz2, idx2 * _CAP + loc2, -1)
  key = (lax.broadcasted_iota(jnp.int32, (_TB2, _E, _CAP), 1) * _CAP +
         lax.broadcasted_iota(jnp.int32, (_TB2, _E, _CAP), 2))
  disp_ref[...] = ((key == k1[:, :, None]) |
                   (key == k2[:, :, None])).astype(jnp.int8)

  @pl.when(i == 0)
  def _laux():
    me = gsum_ref[...] * (1.0 / _T)
    ce = expc.astype(jnp.float32) * (1.0 / _T)
    laux_ref[...] = (jnp.sum(me * ce) * float(_E)).reshape(1, 1)


@functools.partial(
    pl.kernel, mesh=_sc_mesh,
    out_type=jax.ShapeDtypeStruct((_FLAT,), jnp.float32),
    scratch_types=[pltpu.VMEM((_CHUNK,), jnp.float32),
                   pltpu.SemaphoreType.DMA],
)
def _sc_memset(zsrc_hbm, out_hbm, zbuf, sem):
  wid = lax.axis_index("s") * _NC + lax.axis_index("c")
  base = wid * _PER_W
  pltpu.sync_copy(zsrc_hbm, zbuf)

  pltpu.async_copy(zbuf, out_hbm.at[pl.ds(base, _CHUNK)], sem)

  def body(ch, carry):
    @pl.when(ch < _NCH - 1)
    def _start_next():
      pltpu.async_copy(
          zbuf, out_hbm.at[pl.ds(base + (ch + 1) * _CHUNK, _CHUNK)], sem)
    pltpu.make_async_copy(
        zbuf, out_hbm.at[pl.ds(base, _CHUNK)], sem).wait()
    return carry

  lax.fori_loop(0, _NCH, body, 0)


@functools.partial(
    pl.kernel, mesh=_sc_mesh,
    out_type=(),
    scratch_types=[pltpu.VMEM((_TPW,), jnp.int32),
                   pltpu.VMEM((_TPW,), jnp.int32),
                   pltpu.VMEM((_TPW,), jnp.int32),
                   pltpu.VMEM((_TPW,), jnp.int32),
                   pltpu.VMEM((_TPW,), jnp.float32),
                   pltpu.VMEM((_TPW,), jnp.float32),
                   pltpu.VMEM((2, _TPW), jnp.int32),
                   pltpu.VMEM((2, _TPW), jnp.float32),
                   pltpu.SemaphoreType.DMA,
                   pltpu.SemaphoreType.DMA],
)
def _sc_scatter(comb_hbm, idx1_hbm, idx2_hbm, loc1_hbm, loc2_hbm,
                g1_hbm, g2_hbm,
                i1v, i2v, l1v, l2v, g1v, g2v, wv, vv, sem, sem2):
  wid = lax.axis_index("s") * _NC + lax.axis_index("c")
  base = wid * _TPW
  pltpu.sync_copy(idx1_hbm.at[pl.ds(base, _TPW)], i1v)
  pltpu.sync_copy(idx2_hbm.at[pl.ds(base, _TPW)], i2v)
  pltpu.sync_copy(loc1_hbm.at[pl.ds(base, _TPW)], l1v)
  pltpu.sync_copy(loc2_hbm.at[pl.ds(base, _TPW)], l2v)
  pltpu.sync_copy(g1_hbm.at[pl.ds(base, _TPW)], g1v)
  pltpu.sync_copy(g2_hbm.at[pl.ds(base, _TPW)], g2v)

  eps = jnp.float32(jnp.finfo(jnp.float32).eps)
  for j in range(_TPW // _L):
    sl = pl.ds(j * _L, _L)
    i1 = i1v[sl]
    i2 = i2v[sl]
    l1 = l1v[sl]
    l2 = l2v[sl]
    k1 = l1 < _CAP
    k2 = l2 < _CAP
    g1 = jnp.where(k1, g1v[sl], 0.0)
    g2 = jnp.where(k2, g2v[sl], 0.0)
    den = jnp.maximum(g1 + g2, eps)
    tok = base + j * _L + lax.iota(jnp.int32, _L)
    wv[0, sl] = tok * (_E * _CAP) + i1 * _CAP + jnp.where(k1, l1, 0)
    wv[1, sl] = tok * (_E * _CAP) + i2 * _CAP + jnp.where(k2, l2, 0)
    vv[0, sl] = g1 / den
    vv[1, sl] = g2 / den

  c1 = pltpu.async_copy(vv.at[0], comb_hbm.at[wv.at[0]], sem)
  c2 = pltpu.async_copy(vv.at[1], comb_hbm.at[wv.at[1]], sem2)
  c1.wait()
  c2.wait()


def kernel(input, wg):
  x = input.astype(jnp.float32)
  gum = jax.random.gumbel(jax.random.key(42), (_T, _E), jnp.float32)

  zsrc = jnp.zeros((_CHUNK,), jnp.float32)
  comb0 = _sc_memset(zsrc)

  nb1 = _T // _TB1
  vec_i = jax.ShapeDtypeStruct((_T,), jnp.int32)
  vec_f = jax.ShapeDtypeStruct((_T,), jnp.float32)
  row_f = jax.ShapeDtypeStruct((1, _E), jnp.float32)
  row_i = jax.ShapeDtypeStruct((1, _E), jnp.int32)

  vb = pl.BlockSpec((_TB1,), lambda i: (i,))
  rowb = pl.BlockSpec((1, _E), lambda i: (0, 0))

  idx1, idx2, g1, g2, loc1, loc2, gsum, expc = pl.pallas_call(
      _gate_kernel,
      grid=(nb1,),
      in_specs=[
          pl.BlockSpec((_TB1, _D), lambda i: (i, 0)),
          pl.BlockSpec((_D, _E), lambda i: (0, 0)),
          pl.BlockSpec((_TB1, _E), lambda i: (i, 0)),
      ],
      out_specs=[vb, vb, vb, vb, vb, vb, rowb, rowb],
      out_shape=[vec_i, vec_i, vec_f, vec_f, vec_i, vec_i, row_f, row_i],
      scratch_shapes=[pltpu.VMEM((1, _E), jnp.float32),
                      pltpu.VMEM((1, _E), jnp.float32),
                      pltpu.VMEM((_TB1, _TB1), jnp.float32)],
  )(x, wg, gum)

  nb2 = _T // _TB2
  vb2 = pl.BlockSpec((_TB2,), lambda i: (i,))
  rowb2 = pl.BlockSpec((1, _E), lambda i: (0, 0))
  disp, laux, loc2f = pl.pallas_call(
      _disp_kernel,
      grid=(nb2,),
      in_specs=[vb2, vb2, vb2, vb2, vb2, vb2, rowb2, rowb2],
      out_specs=[
          pl.BlockSpec((_TB2, _E, _CAP), lambda i: (i, 0, 0)),
          pl.BlockSpec((1, 1), lambda i: (0, 0)),
          vb2,
      ],
      out_shape=[
          jax.ShapeDtypeStruct((_T, _E, _CAP), jnp.int8),
          jax.ShapeDtypeStruct((1, 1), jnp.float32),
          jax.ShapeDtypeStruct((_T,), jnp.int32),
      ],
  )(idx1, idx2, loc1, loc2, g1, g2, expc, gsum)

  cref = jax.new_ref(comb0)
  _sc_scatter(cref, idx1, idx2, loc1, loc2f, g1, g2)
  comb = jax.freeze(cref).reshape(_T, _E, _CAP)

  dispb = disp.view(jnp.bool_)
  return (laux.reshape(()), comb, dispb, expc.reshape(_E))
